# baseline (speedup 1.0000x reference)
import functools

import jax
import jax.numpy as jnp
from jax import lax
from jax.experimental import pallas as pl
from jax.experimental.pallas import tpu as pltpu
from jax.experimental.pallas import tpu_sc as plsc

NSTATIONS = 50000
S_PAD = 50176
DUMP = S_PAD - 128
NC = 2
NS = 16
NW = NC * NS
SLICE = S_PAD // NS
C_SUB = 1280
NV = C_SUB // 16
PADL = 8
CAPC = 1440


def _sc_accumulate_body(p_hbm, t_hbm, s_hbm, out_hbm,
                        s_b, t_b, p_b, ids_b, ec_b, et_b, et2_b, er2_b,
                        dc_b, dt_b, dt2_b, dr2_b, fio_v, z_v,
                        tbl_c, tbl_t, tbl_t2, tbl_r2,
                        ssem, lsem):
    cid = lax.axis_index("c")
    sid = lax.axis_index("s")
    wid = cid * NS + sid

    zeros16 = jnp.zeros((16,), jnp.float32)
    idx15 = jnp.full((16,), 15, jnp.int32)
    fio_v[...] = (lax.iota(jnp.int32, 16) + 1).astype(jnp.float32)

    def _zfill(i, _):
        z_v[pl.ds(i * 16, 16)] = zeros16
        return 0
    lax.fori_loop(0, SLICE // 16, _zfill, 0)

    def _efill(i, _):
        for k in (0, 1):
            ec_b[k][pl.ds(i * 16, 16)] = zeros16
            et_b[k][pl.ds(i * 16, 16)] = zeros16
            et2_b[k][pl.ds(i * 16, 16)] = zeros16
            er2_b[k][pl.ds(i * 16, 16)] = zeros16
        return 0
    lax.fori_loop(0, CAPC // 16, _efill, 0)

    base = sid * SLICE
    pltpu.sync_copy(z_v, tbl_c.at[pl.ds(base, SLICE)])
    pltpu.sync_copy(z_v, tbl_t.at[pl.ds(base, SLICE)])
    pltpu.sync_copy(z_v, tbl_t2.at[pl.ds(base, SLICE)])
    pltpu.sync_copy(z_v, tbl_r2.at[pl.ds(base, SLICE)])
    plsc.subcore_barrier()

    n = p_hbm.shape[0]
    nsub = n // C_SUB
    nsubw = (nsub + NW - 1) // NW

    def _issue_load(slot, g):
        sc = wid + g * NW

        @pl.when(sc < nsub)
        def _():
            e0 = sc * C_SUB
            pltpu.async_copy(p_hbm.at[pl.ds(e0, C_SUB)], p_b[slot], sem=lsem[slot])
            pltpu.async_copy(t_hbm.at[pl.ds(e0, C_SUB)], t_b[slot], sem=lsem[slot])
            pltpu.async_copy(s_hbm.at[pl.ds(e0, C_SUB)],
                             s_b[slot].at[pl.ds(0, C_SUB)], sem=lsem[slot])

    def _wait_load(slot, g):
        sc = wid + g * NW

        @pl.when(sc < nsub)
        def _():
            pltpu.make_async_copy(p_hbm.at[pl.ds(0, C_SUB)], p_b[slot], lsem[slot]).wait()
            pltpu.make_async_copy(t_hbm.at[pl.ds(0, C_SUB)], t_b[slot], lsem[slot]).wait()
            pltpu.make_async_copy(s_hbm.at[pl.ds(0, C_SUB)],
                                  s_b[slot].at[pl.ds(0, C_SUB)], lsem[slot]).wait()

    def _process_pair(sl0, sl1, g):
        slots = (sl0, sl1)
        for k in (0, 1):
            s_b[slots[k]][pl.ds(C_SUB, 16)] = jnp.full((16,), -1, jnp.int32)
            ec_b[k][pl.ds(0, 16)] = zeros16
            et_b[k][pl.ds(0, 16)] = zeros16
            et2_b[k][pl.ds(0, 16)] = zeros16
            er2_b[k][pl.ds(0, 16)] = zeros16

        cur = [jnp.full((16,), PADL, jnp.int32) for _ in range(2)]
        ct = [zeros16, zeros16]
        ct2 = [zeros16, zeros16]
        cr2 = [zeros16, zeros16]
        for i in range(NV):
            b = i * 16
            for k in (0, 1):
                sv = s_b[slots[k]]
                tv = t_b[slots[k]]
                pv = p_b[slots[k]]
                scur = sv[pl.ds(b, 16)]
                snext = sv[pl.ds(b + 1, 16)]
                m = scur != snext
                tt = tv[pl.ds(b, 16)]
                pp = pv[pl.ds(b, 16)]
                d = tt - pp
                t2x = tt * tt
                r2x = d * d
                incl = plsc.cumsum(m.astype(jnp.int32))
                cs_t = plsc.cumsum(tt) + ct[k]
                cs_t2 = plsc.cumsum(t2x) + ct2[k]
                cs_r2 = plsc.cumsum(r2x) + cr2[k]
                ct[k] = cs_t[idx15]
                ct2[k] = cs_t2[idx15]
                cr2[k] = cs_r2[idx15]
                cs_c = fio_v[...] + jnp.float32(b)
                dest = cur[k] + incl - 1
                plsc.store_scatter(ids_b[k], [dest], scur, mask=m)
                plsc.store_scatter(ec_b[k], [dest], cs_c, mask=m)
                plsc.store_scatter(et_b[k], [dest], cs_t, mask=m)
                plsc.store_scatter(et2_b[k], [dest], cs_t2, mask=m)
                plsc.store_scatter(er2_b[k], [dest], cs_r2, mask=m)
                cur[k] = cur[k] + incl[idx15]

        dump16 = jnp.full((16,), DUMP, jnp.int32)
        for k in (0, 1):
            base_sent = cur[k] + lax.iota(jnp.int32, 16)
            for r in range(8):
                plsc.store_scatter(ids_b[k], [base_sent + r * 16], dump16)

        for k in (0, 1):
            sc = wid + (g + k) * NW
            kcnt = jnp.max(cur[k]) - PADL
            nstr = jnp.where(sc < nsub, (kcnt + 127) // 128, 0)
            d_c, d_t, d_t2, d_r2 = dc_b[k], dt_b[k], dt2_b[k], dr2_b[k]
            e_c, e_t, e_t2, e_r2 = ec_b[k], et_b[k], et2_b[k], er2_b[k]
            ids_c = ids_b[k]
            sem = ssem[k]

            def _scat(j, _):
                for l in range(8):
                    bb = PADL + j * 128 + l * 16
                    w = pl.ds(bb, 16)
                    wp = pl.ds(bb - 1, 16)
                    d_c[w] = e_c[w] - e_c[wp]
                    d_t[w] = e_t[w] - e_t[wp]
                    d_t2[w] = e_t2[w] - e_t2[wp]
                    d_r2[w] = e_r2[w] - e_r2[wp]
                rs = pl.ds(PADL + j * 128, 128)
                idx = ids_c.at[rs]
                c0 = pltpu.async_copy(d_c.at[rs], tbl_c.at[idx], add=True, sem=sem)
                c1 = pltpu.async_copy(d_t.at[rs], tbl_t.at[idx], add=True, sem=sem)
                c2 = pltpu.async_copy(d_t2.at[rs], tbl_t2.at[idx], add=True, sem=sem)
                c3 = pltpu.async_copy(d_r2.at[rs], tbl_r2.at[idx], add=True, sem=sem)
                c0.wait()
                c1.wait()
                c2.wait()
                c3.wait()
                return 0

            lax.fori_loop(0, nstr, _scat, 0)

    _issue_load(0, 0)
    _issue_load(1, 1)

    def _quad(it, _):
        g = 4 * it
        _issue_load(2, g + 2)
        _issue_load(3, g + 3)
        _wait_load(0, g)
        _wait_load(1, g + 1)
        _process_pair(0, 1, g)
        _issue_load(0, g + 4)
        _issue_load(1, g + 5)
        _wait_load(2, g + 2)
        _wait_load(3, g + 3)
        _process_pair(2, 3, g + 2)
        return 0

    lax.fori_loop(0, (nsubw + 3) // 4, _quad, 0)
    plsc.subcore_barrier()

    out0 = cid * (4 * S_PAD) + base
    for kk, tbl in enumerate((tbl_c, tbl_t, tbl_t2, tbl_r2)):
        pltpu.sync_copy(tbl.at[pl.ds(base, SLICE)], z_v)
        pltpu.sync_copy(z_v, out_hbm.at[pl.ds(out0 + kk * S_PAD, SLICE)])


@functools.partial(
    pl.kernel,
    out_type=jax.ShapeDtypeStruct((2 * 4 * S_PAD,), jnp.float32),
    mesh=plsc.VectorSubcoreMesh(core_axis_name="c", subcore_axis_name="s",
                                num_cores=NC, num_subcores=NS),
    compiler_params=pltpu.CompilerParams(needs_layout_passes=False),
    scratch_types=[
        [pltpu.VMEM((C_SUB + 16,), jnp.int32)] * 4,
        [pltpu.VMEM((C_SUB,), jnp.float32)] * 4,
        [pltpu.VMEM((C_SUB,), jnp.float32)] * 4,
        [pltpu.VMEM((CAPC,), jnp.int32)] * 2,
        [pltpu.VMEM((CAPC,), jnp.float32)] * 2,
        [pltpu.VMEM((CAPC,), jnp.float32)] * 2,
        [pltpu.VMEM((CAPC,), jnp.float32)] * 2,
        [pltpu.VMEM((CAPC,), jnp.float32)] * 2,
        [pltpu.VMEM((CAPC,), jnp.float32)] * 2,
        [pltpu.VMEM((CAPC,), jnp.float32)] * 2,
        [pltpu.VMEM((CAPC,), jnp.float32)] * 2,
        [pltpu.VMEM((CAPC,), jnp.float32)] * 2,
        pltpu.VMEM((16,), jnp.float32),
        pltpu.VMEM((SLICE,), jnp.float32),
        pltpu.VMEM_SHARED((S_PAD,), jnp.float32),
        pltpu.VMEM_SHARED((S_PAD,), jnp.float32),
        pltpu.VMEM_SHARED((S_PAD,), jnp.float32),
        pltpu.VMEM_SHARED((S_PAD,), jnp.float32),
        [pltpu.SemaphoreType.DMA] * 2,
        [pltpu.SemaphoreType.DMA] * 4,
    ],
)
def _sc_accumulate(*args):
    _sc_accumulate_body(*args)


def _finalize_body(x_ref, o_ref):
    c = x_ref[0] + x_ref[4]
    st = x_ref[1] + x_ref[5]
    st2 = x_ref[2] + x_ref[6]
    sr = x_ref[3] + x_ref[7]
    rows = S_PAD // 128
    gidx = (lax.broadcasted_iota(jnp.int32, (rows, 128), 0) * 128
            + lax.broadcasted_iota(jnp.int32, (rows, 128), 1))
    valid = gidx < NSTATIONS
    cs = jnp.maximum(c, 1.0)
    ss_tot = st2 - st * st / cs
    mse = sr / cs
    ss_tot_safe = jnp.where(ss_tot > 1e-8, ss_tot, 1.0)
    r2 = 1.0 - sr / ss_tot_safe
    r2 = jnp.clip(r2, -1.0, 1.0)
    loss_r2 = 1.0 - r2
    use_mse = (c < 5.0) | (ss_tot <= 1e-8)
    sl = jnp.where(use_mse, mse, loss_r2)
    keep = valid & (c > 0.0)
    sl = jnp.where(keep, sl, 0.0)
    n_uniq = jnp.sum(keep.astype(jnp.float32))
    val = jnp.sum(sl) / jnp.maximum(n_uniq, 1.0)
    o_ref[...] = val[None, None]


def kernel(predictions, targets, station_ids):
    s1 = station_ids.astype(jnp.int32)
    partials = _sc_accumulate(predictions, targets, s1)
    x = partials.reshape(8, S_PAD // 128, 128)
    out = pl.pallas_call(
        _finalize_body,
        out_shape=jax.ShapeDtypeStruct((1, 1), jnp.float32),
    )(x)
    return out.reshape(())


# speedup vs baseline: 1.1886x; 1.1886x over previous
import functools

import jax
import jax.numpy as jnp
from jax import lax
from jax.experimental import pallas as pl
from jax.experimental.pallas import tpu as pltpu
from jax.experimental.pallas import tpu_sc as plsc

NSTATIONS = 50000
S_PAD = 50176
DUMP = S_PAD - 128
NC = 2
NS = 16
NW = NC * NS
SLICE = S_PAD // NS
C_SUB = 1280
NV = C_SUB // 16
PADL = 8
CAPC = 1440


def _sc_accumulate_body(p_hbm, t_hbm, s_hbm, out_hbm,
                        s_b, t_b, p_b, ids_b, ec_b, et_b, et2_b, er2_b,
                        dc_b, dt_b, dt2_b, dr2_b, fio_v, z_v,
                        tbl_c, tbl_t, tbl_t2, tbl_r2,
                        ssem, lsem):
    cid = lax.axis_index("c")
    sid = lax.axis_index("s")
    wid = cid * NS + sid

    zeros16 = jnp.zeros((16,), jnp.float32)
    idx15 = jnp.full((16,), 15, jnp.int32)
    fio_v[...] = (lax.iota(jnp.int32, 16) + 1).astype(jnp.float32)

    def _zfill(i, _):
        z_v[pl.ds(i * 16, 16)] = zeros16
        return 0
    lax.fori_loop(0, SLICE // 16, _zfill, 0)

    def _efill(i, _):
        ec_b[0][pl.ds(i * 16, 16)] = zeros16
        et_b[0][pl.ds(i * 16, 16)] = zeros16
        et2_b[0][pl.ds(i * 16, 16)] = zeros16
        er2_b[0][pl.ds(i * 16, 16)] = zeros16
        return 0
    lax.fori_loop(0, CAPC // 16, _efill, 0)

    base = sid * SLICE
    pltpu.sync_copy(z_v, tbl_c.at[pl.ds(base, SLICE)])
    pltpu.sync_copy(z_v, tbl_t.at[pl.ds(base, SLICE)])
    pltpu.sync_copy(z_v, tbl_t2.at[pl.ds(base, SLICE)])
    pltpu.sync_copy(z_v, tbl_r2.at[pl.ds(base, SLICE)])
    plsc.subcore_barrier()

    n = p_hbm.shape[0]
    nsub = n // C_SUB
    nsubw = (nsub + NW - 1) // NW

    def _issue_load(slot, g):
        sc = wid + g * NW

        @pl.when(sc < nsub)
        def _():
            e0 = sc * C_SUB
            pltpu.async_copy(p_hbm.at[pl.ds(e0, C_SUB)], p_b[slot], sem=lsem[slot])
            pltpu.async_copy(t_hbm.at[pl.ds(e0, C_SUB)], t_b[slot], sem=lsem[slot])
            pltpu.async_copy(s_hbm.at[pl.ds(e0, C_SUB)],
                             s_b[slot].at[pl.ds(0, C_SUB)], sem=lsem[slot])

    def _wait_load(slot, g):
        sc = wid + g * NW

        @pl.when(sc < nsub)
        def _():
            pltpu.make_async_copy(p_hbm.at[pl.ds(0, C_SUB)], p_b[slot], lsem[slot]).wait()
            pltpu.make_async_copy(t_hbm.at[pl.ds(0, C_SUB)], t_b[slot], lsem[slot]).wait()
            pltpu.make_async_copy(s_hbm.at[pl.ds(0, C_SUB)],
                                  s_b[slot].at[pl.ds(0, C_SUB)], lsem[slot]).wait()

    def _process(slot, g):
        sc = wid + g * NW

        @pl.when(sc < nsub)
        def _():
            sv = s_b[slot]
            tv = t_b[slot]
            pv = p_b[slot]
            ids_c, e_c, e_t, e_t2, e_r2 = ids_b[0], ec_b[0], et_b[0], et2_b[0], er2_b[0]
            d_c, d_t, d_t2, d_r2 = dc_b[0], dt_b[0], dt2_b[0], dr2_b[0]
            sem = ssem[0]
            sv[pl.ds(C_SUB, 16)] = jnp.full((16,), -1, jnp.int32)
            e_c[pl.ds(0, 16)] = zeros16
            e_t[pl.ds(0, 16)] = zeros16
            e_t2[pl.ds(0, 16)] = zeros16
            e_r2[pl.ds(0, 16)] = zeros16

            cur = jnp.int32(PADL)
            ct = zeros16
            ct2 = zeros16
            cr2 = zeros16
            for i in range(NV):
                b = i * 16
                scur = sv[pl.ds(b, 16)]
                snext = sv[pl.ds(b + 1, 16)]
                m = scur != snext
                tt = tv[pl.ds(b, 16)]
                pp = pv[pl.ds(b, 16)]
                d = tt - pp
                t2x = tt * tt
                r2x = d * d
                cs_t = plsc.cumsum(tt) + ct
                cs_t2 = plsc.cumsum(t2x) + ct2
                cs_r2 = plsc.cumsum(r2x) + cr2
                ct = cs_t[idx15]
                ct2 = cs_t2[idx15]
                cr2 = cs_r2[idx15]
                cs_c = fio_v[...] + jnp.float32(b)
                win = pl.ds(cur, 16)
                plsc.store_compressed(ids_c.at[win], scur, mask=m)
                plsc.store_compressed(e_c.at[win], cs_c, mask=m)
                plsc.store_compressed(e_t.at[win], cs_t, mask=m)
                plsc.store_compressed(e_t2.at[win], cs_t2, mask=m)
                plsc.store_compressed(e_r2.at[win], cs_r2, mask=m)
                pc = plsc.all_reduce_population_count(m)
                cur = cur + pc[0]

            dump16 = jnp.full((16,), DUMP, jnp.int32)
            for r in range(8):
                ids_c[pl.ds(cur + r * 16, 16)] = dump16

            nstr = (cur - PADL + 127) // 128

            def _scat(j, _):
                for l in range(8):
                    bb = PADL + j * 128 + l * 16
                    w = pl.ds(bb, 16)
                    wp = pl.ds(bb - 1, 16)
                    d_c[w] = e_c[w] - e_c[wp]
                    d_t[w] = e_t[w] - e_t[wp]
                    d_t2[w] = e_t2[w] - e_t2[wp]
                    d_r2[w] = e_r2[w] - e_r2[wp]
                rs = pl.ds(PADL + j * 128, 128)
                idx = ids_c.at[rs]
                c0 = pltpu.async_copy(d_c.at[rs], tbl_c.at[idx], add=True, sem=sem)
                c1 = pltpu.async_copy(d_t.at[rs], tbl_t.at[idx], add=True, sem=sem)
                c2 = pltpu.async_copy(d_t2.at[rs], tbl_t2.at[idx], add=True, sem=sem)
                c3 = pltpu.async_copy(d_r2.at[rs], tbl_r2.at[idx], add=True, sem=sem)
                c0.wait()
                c1.wait()
                c2.wait()
                c3.wait()
                return 0

            lax.fori_loop(0, nstr, _scat, 0)

    _issue_load(0, 0)

    def _pair(it, _):
        g = 2 * it
        _issue_load(1, g + 1)
        _wait_load(0, g)
        _process(0, g)
        _issue_load(0, g + 2)
        _wait_load(1, g + 1)
        _process(1, g + 1)
        return 0

    lax.fori_loop(0, (nsubw + 1) // 2, _pair, 0)
    plsc.subcore_barrier()

    out0 = cid * (4 * S_PAD) + base
    for kk, tbl in enumerate((tbl_c, tbl_t, tbl_t2, tbl_r2)):
        pltpu.sync_copy(tbl.at[pl.ds(base, SLICE)], z_v)
        pltpu.sync_copy(z_v, out_hbm.at[pl.ds(out0 + kk * S_PAD, SLICE)])


@functools.partial(
    pl.kernel,
    out_type=jax.ShapeDtypeStruct((2 * 4 * S_PAD,), jnp.float32),
    mesh=plsc.VectorSubcoreMesh(core_axis_name="c", subcore_axis_name="s",
                                num_cores=NC, num_subcores=NS),
    compiler_params=pltpu.CompilerParams(needs_layout_passes=False),
    scratch_types=[
        [pltpu.VMEM((C_SUB + 16,), jnp.int32)] * 2,
        [pltpu.VMEM((C_SUB,), jnp.float32)] * 2,
        [pltpu.VMEM((C_SUB,), jnp.float32)] * 2,
        [pltpu.VMEM((CAPC,), jnp.int32)] * 1,
        [pltpu.VMEM((CAPC,), jnp.float32)] * 1,
        [pltpu.VMEM((CAPC,), jnp.float32)] * 1,
        [pltpu.VMEM((CAPC,), jnp.float32)] * 1,
        [pltpu.VMEM((CAPC,), jnp.float32)] * 1,
        [pltpu.VMEM((CAPC,), jnp.float32)] * 1,
        [pltpu.VMEM((CAPC,), jnp.float32)] * 1,
        [pltpu.VMEM((CAPC,), jnp.float32)] * 1,
        [pltpu.VMEM((CAPC,), jnp.float32)] * 1,
        pltpu.VMEM((16,), jnp.float32),
        pltpu.VMEM((SLICE,), jnp.float32),
        pltpu.VMEM_SHARED((S_PAD,), jnp.float32),
        pltpu.VMEM_SHARED((S_PAD,), jnp.float32),
        pltpu.VMEM_SHARED((S_PAD,), jnp.float32),
        pltpu.VMEM_SHARED((S_PAD,), jnp.float32),
        [pltpu.SemaphoreType.DMA] * 1,
        [pltpu.SemaphoreType.DMA] * 2,
    ],
)
def _sc_accumulate(*args):
    _sc_accumulate_body(*args)


def _finalize_body(x_ref, o_ref):
    c = x_ref[0] + x_ref[4]
    st = x_ref[1] + x_ref[5]
    st2 = x_ref[2] + x_ref[6]
    sr = x_ref[3] + x_ref[7]
    rows = S_PAD // 128
    gidx = (lax.broadcasted_iota(jnp.int32, (rows, 128), 0) * 128
            + lax.broadcasted_iota(jnp.int32, (rows, 128), 1))
    valid = gidx < NSTATIONS
    cs = jnp.maximum(c, 1.0)
    ss_tot = st2 - st * st / cs
    mse = sr / cs
    ss_tot_safe = jnp.where(ss_tot > 1e-8, ss_tot, 1.0)
    r2 = 1.0 - sr / ss_tot_safe
    r2 = jnp.clip(r2, -1.0, 1.0)
    loss_r2 = 1.0 - r2
    use_mse = (c < 5.0) | (ss_tot <= 1e-8)
    sl = jnp.where(use_mse, mse, loss_r2)
    keep = valid & (c > 0.0)
    sl = jnp.where(keep, sl, 0.0)
    n_uniq = jnp.sum(keep.astype(jnp.float32))
    val = jnp.sum(sl) / jnp.maximum(n_uniq, 1.0)
    o_ref[...] = val[None, None]


def kernel(predictions, targets, station_ids):
    s1 = station_ids.astype(jnp.int32)
    partials = _sc_accumulate(predictions, targets, s1)
    x = partials.reshape(8, S_PAD // 128, 128)
    out = pl.pallas_call(
        _finalize_body,
        out_shape=jax.ShapeDtypeStruct((1, 1), jnp.float32),
    )(x)
    return out.reshape(())
